# Initial kernel scaffold; baseline (speedup 1.0000x reference)
import functools

import jax
import jax.numpy as jnp
from jax import lax
from jax.experimental import pallas as pl
from jax.experimental.pallas import tpu as pltpu
from jax.experimental.pallas import tpu_sc as plsc

B = 16384
L = 50
D = 32

NC = 2
NS = 16
NW = NC * NS
BW = B // NW
RW = BW * L
SZ = 128
NSUB = RW // SZ
NBUF = 2
UID_SUB = BW // SZ


def _sc_body(uid_hbm, hist_hbm, wu_hbm, wi_hbm, uid_out, sums_out,
             uidx, hidx, sidx, gbufs, zbuf, acc, gsems):
    c = lax.axis_index("c")
    s = lax.axis_index("s")
    w = c * NS + s

    pltpu.sync_copy(uid_hbm.at[pl.ds(w * UID_SUB, UID_SUB)], uidx)
    pltpu.sync_copy(hist_hbm.at[pl.ds(w * NSUB, NSUB)], hidx)

    base_row = s * BW

    def build(t, _):
        for v in range(SZ // 16):
            pos = (t * SZ + v * 16) + lax.broadcasted_iota(jnp.int32, (16,), 0)
            sidx[t, pl.ds(v * 16, 16)] = pos // L + base_row
        return 0

    lax.fori_loop(0, NSUB, build, 0)

    zf = jnp.zeros((16,), jnp.float32)

    def zero(r, _):
        zbuf[r, pl.ds(0, 16)] = zf
        zbuf[r, pl.ds(16, 16)] = zf
        return 0

    lax.fori_loop(0, SZ, zero, 0)
    for k in range(BW // SZ):
        pltpu.sync_copy(zbuf, acc.at[pl.ds(base_row + k * SZ, SZ)])

    for k in range(UID_SUB):
        b = k % NBUF
        pltpu.async_copy(wu_hbm.at[uidx.at[k]], gbufs[b], gsems[b])
        if k >= 1:
            pb = (k - 1) % NBUF
            pltpu.make_async_copy(wu_hbm.at[pl.ds(0, SZ)], gbufs[pb],
                                  gsems[pb]).wait()
            pltpu.sync_copy(gbufs[pb],
                            uid_out.at[pl.ds(w * BW + (k - 1) * SZ, SZ)])
    lb = (UID_SUB - 1) % NBUF
    pltpu.make_async_copy(wu_hbm.at[pl.ds(0, SZ)], gbufs[lb], gsems[lb]).wait()
    pltpu.sync_copy(gbufs[lb],
                    uid_out.at[pl.ds(w * BW + (UID_SUB - 1) * SZ, SZ)])

    for b in range(NBUF):
        pltpu.async_copy(wi_hbm.at[hidx.at[b]], gbufs[b], gsems[b])

    def step(ti, _):
        for b in range(NBUF):
            t = ti * NBUF + b
            pltpu.make_async_copy(wi_hbm.at[pl.ds(0, SZ)], gbufs[b],
                                  gsems[b]).wait()
            pltpu.sync_copy(gbufs[b], acc.at[sidx.at[t]], add=True)
            nt = t + NBUF

            @pl.when(nt < NSUB)
            def _():
                pltpu.async_copy(wi_hbm.at[hidx.at[nt]], gbufs[b], gsems[b])
        return 0

    lax.fori_loop(0, NSUB // NBUF, step, 0)

    pltpu.sync_copy(acc.at[pl.ds(base_row, BW)],
                    sums_out.at[pl.ds(w * BW, BW)])


@jax.jit
def _sc_lookup(uid2, hist2, W_uid, W_item):
    mesh = plsc.VectorSubcoreMesh(core_axis_name="c", subcore_axis_name="s")
    return pl.kernel(
        _sc_body,
        out_type=(
            jax.ShapeDtypeStruct((B, D), jnp.float32),
            jax.ShapeDtypeStruct((B, D), jnp.float32),
        ),
        mesh=mesh,
        scratch_types=[
            pltpu.VMEM((UID_SUB, SZ), jnp.int32),
            pltpu.VMEM((NSUB, SZ), jnp.int32),
            pltpu.VMEM((NSUB, SZ), jnp.int32),
            [pltpu.VMEM((SZ, D), jnp.float32) for _ in range(NBUF)],
            pltpu.VMEM((SZ, D), jnp.float32),
            pltpu.VMEM_SHARED((NS * BW, D), jnp.float32),
            [pltpu.SemaphoreType.DMA for _ in range(NBUF)],
        ],
    )(uid2, hist2, W_uid, W_item)


def _tc_body(hist_ref, sums_ref, w0_ref, out_ref):
    hist = hist_ref[...]
    cnt = jnp.sum((hist != 0).astype(jnp.float32), axis=1, keepdims=True)
    w0 = w0_ref[...]
    corrected = sums_ref[...] - (float(L) - cnt) * w0
    pooled = corrected / jnp.maximum(cnt, 1.0)
    out_ref[...] = jnp.where(cnt > 0.0, pooled, 0.0)


@jax.jit
def _tc_combine(hist_item, sums, w0):
    blk = 2048
    return pl.pallas_call(
        _tc_body,
        grid=(B // blk,),
        in_specs=[
            pl.BlockSpec((blk, L), lambda i: (i, 0)),
            pl.BlockSpec((blk, D), lambda i: (i, 0)),
            pl.BlockSpec((1, D), lambda i: (0, 0)),
        ],
        out_specs=pl.BlockSpec((blk, D), lambda i: (i, 0)),
        out_shape=jax.ShapeDtypeStruct((B, D), jnp.float32),
    )(hist_item, sums, w0)


def kernel(uid, hist_item, W_uid, W_item):
    uid2 = uid.astype(jnp.int32).reshape(NW * UID_SUB, SZ)
    hist2 = hist_item.astype(jnp.int32).reshape(NW * NSUB, SZ)
    uid_emb, sums = _sc_lookup(uid2, hist2, W_uid, W_item)
    w0 = lax.slice(W_item, (0, 0), (1, D))
    pooled = _tc_combine(hist_item, sums, w0)
    return (uid_emb, pooled)


# speedup vs baseline: 1.6424x; 1.6424x over previous
import jax
import jax.numpy as jnp
from jax import lax
from jax.experimental import pallas as pl
from jax.experimental.pallas import tpu as pltpu
from jax.experimental.pallas import tpu_sc as plsc

B = 16384
L = 50
D = 32

NC = 2
NS = 16
NW = NC * NS
BW = B // NW
RW = BW * L
SZ = 128
NSUB = RW // SZ
NBUF = 2
UID_SUB = BW // SZ


def _sc_body(uid_hbm, hist_hbm, sidx_hbm, wu_hbm, wi_hbm, uid_out, sums_out,
             uidx, hidx, sidx, gbufs, zbuf, acc, gsems):
    c = lax.axis_index("c")
    s = lax.axis_index("s")
    w = c * NS + s
    base_row = s * BW

    pltpu.sync_copy(uid_hbm.at[pl.ds(w * UID_SUB, UID_SUB)], uidx)
    pltpu.sync_copy(hist_hbm.at[pl.ds(w * NSUB, NSUB)], hidx)
    pltpu.sync_copy(sidx_hbm.at[pl.ds(w * NSUB, NSUB)], sidx)

    zf = jnp.zeros((16,), jnp.float32)

    def zero(r, _):
        zbuf[r, pl.ds(0, 16)] = zf
        zbuf[r, pl.ds(16, 16)] = zf
        return 0

    lax.fori_loop(0, SZ, zero, 0)
    for k in range(BW // SZ):
        pltpu.sync_copy(zbuf, acc.at[pl.ds(base_row + k * SZ, SZ)])

    for k in range(UID_SUB):
        b = k % NBUF
        pltpu.async_copy(wu_hbm.at[uidx.at[k]], gbufs[b], gsems[b])
        if k >= 1:
            pb = (k - 1) % NBUF
            pltpu.make_async_copy(wu_hbm.at[pl.ds(0, SZ)], gbufs[pb],
                                  gsems[pb]).wait()
            pltpu.sync_copy(gbufs[pb],
                            uid_out.at[pl.ds(w * BW + (k - 1) * SZ, SZ)])
    lb = (UID_SUB - 1) % NBUF
    pltpu.make_async_copy(wu_hbm.at[pl.ds(0, SZ)], gbufs[lb], gsems[lb]).wait()
    pltpu.sync_copy(gbufs[lb],
                    uid_out.at[pl.ds(w * BW + (UID_SUB - 1) * SZ, SZ)])

    for b in range(NBUF):
        pltpu.async_copy(wi_hbm.at[hidx.at[b]], gbufs[b], gsems[b])

    def step(ti, _):
        for b in range(NBUF):
            t = ti * NBUF + b
            pltpu.make_async_copy(wi_hbm.at[pl.ds(0, SZ)], gbufs[b],
                                  gsems[b]).wait()
            pltpu.sync_copy(gbufs[b], acc.at[sidx.at[t]], add=True)
            nt = t + NBUF

            @pl.when(nt < NSUB)
            def _():
                pltpu.async_copy(wi_hbm.at[hidx.at[nt]], gbufs[b], gsems[b])
        return 0

    lax.fori_loop(0, NSUB // NBUF, step, 0)

    pltpu.sync_copy(acc.at[pl.ds(base_row, BW)],
                    sums_out.at[pl.ds(w * BW, BW)])


def _sc_lookup(uid2, hist2, sidx2, W_uid, W_item):
    mesh = plsc.VectorSubcoreMesh(core_axis_name="c", subcore_axis_name="s")
    return pl.kernel(
        _sc_body,
        out_type=(
            jax.ShapeDtypeStruct((B, D), jnp.float32),
            jax.ShapeDtypeStruct((B, D), jnp.float32),
        ),
        mesh=mesh,
        compiler_params=pltpu.CompilerParams(use_tc_tiling_on_sc=False),
        scratch_types=[
            pltpu.VMEM((UID_SUB, SZ), jnp.int32),
            pltpu.VMEM((NSUB, SZ), jnp.int32),
            pltpu.VMEM((NSUB, SZ), jnp.int32),
            [pltpu.VMEM((SZ, D), jnp.float32) for _ in range(NBUF)],
            pltpu.VMEM((SZ, D), jnp.float32),
            pltpu.VMEM_SHARED((NS * BW, D), jnp.float32),
            [pltpu.SemaphoreType.DMA for _ in range(NBUF)],
        ],
    )(uid2, hist2, sidx2, W_uid, W_item)


def _tc_body(hist_ref, sums_ref, w0_ref, out_ref):
    hist = hist_ref[...]
    cnt = jnp.sum((hist != 0).astype(jnp.float32), axis=1, keepdims=True)
    w0 = w0_ref[...]
    corrected = sums_ref[...] - (float(L) - cnt) * w0
    pooled = corrected / jnp.maximum(cnt, 1.0)
    out_ref[...] = jnp.where(cnt > 0.0, pooled, 0.0)


def _tc_combine(hist_item, sums, w0):
    blk = 2048
    return pl.pallas_call(
        _tc_body,
        grid=(B // blk,),
        in_specs=[
            pl.BlockSpec((blk, L), lambda i: (i, 0)),
            pl.BlockSpec((blk, D), lambda i: (i, 0)),
            pl.BlockSpec((1, D), lambda i: (0, 0)),
        ],
        out_specs=pl.BlockSpec((blk, D), lambda i: (i, 0)),
        out_shape=jax.ShapeDtypeStruct((B, D), jnp.float32),
    )(hist_item, sums, w0)


def kernel(uid, hist_item, W_uid, W_item):
    uid2 = uid.astype(jnp.int32).reshape(NW * UID_SUB, SZ)
    hist2 = hist_item.astype(jnp.int32).reshape(NW * NSUB, SZ)
    sidx2 = ((lax.iota(jnp.int32, B * L) // L) % (NS * BW)).reshape(
        NW * NSUB, SZ)
    uid_emb, sums = _sc_lookup(uid2, hist2, sidx2, W_uid, W_item)
    w0 = lax.slice(W_item, (0, 0), (1, D))
    pooled = _tc_combine(hist_item, sums, w0)
    return (uid_emb, pooled)


# speedup vs baseline: 1.6986x; 1.0342x over previous
import jax
import jax.numpy as jnp
from jax import lax
from jax.experimental import pallas as pl
from jax.experimental.pallas import tpu as pltpu
from jax.experimental.pallas import tpu_sc as plsc

B = 16384
L = 50
D = 32

NC = 2
NS = 16
NW = NC * NS
BW = B // NW
RW = BW * L
SZ = 128
NSUB = RW // SZ
NBUF = 8
UID_SUB = BW // SZ


def _sc_body(uid_hbm, hist_hbm, sidx_hbm, wu_hbm, wi_hbm, uid_out, sums_out,
             uidx, hidx, sidx, gbufs, zbuf, acc, gsems):
    c = lax.axis_index("c")
    s = lax.axis_index("s")
    w = c * NS + s
    base_row = s * BW

    pltpu.sync_copy(uid_hbm.at[pl.ds(w * UID_SUB, UID_SUB)], uidx)
    pltpu.sync_copy(hist_hbm.at[pl.ds(w * NSUB, NSUB)], hidx)
    pltpu.sync_copy(sidx_hbm.at[pl.ds(w * NSUB, NSUB)], sidx)

    zf = jnp.zeros((16,), jnp.float32)

    def zero(r, _):
        zbuf[r, pl.ds(0, 16)] = zf
        zbuf[r, pl.ds(16, 16)] = zf
        return 0

    lax.fori_loop(0, SZ, zero, 0)
    for k in range(BW // SZ):
        pltpu.sync_copy(zbuf, acc.at[pl.ds(base_row + k * SZ, SZ)])

    for k in range(UID_SUB):
        b = k % NBUF
        pltpu.async_copy(wu_hbm.at[uidx.at[k]], gbufs[b], gsems[b])
        if k >= 1:
            pb = (k - 1) % NBUF
            pltpu.make_async_copy(wu_hbm.at[pl.ds(0, SZ)], gbufs[pb],
                                  gsems[pb]).wait()
            pltpu.sync_copy(gbufs[pb],
                            uid_out.at[pl.ds(w * BW + (k - 1) * SZ, SZ)])
    lb = (UID_SUB - 1) % NBUF
    pltpu.make_async_copy(wu_hbm.at[pl.ds(0, SZ)], gbufs[lb], gsems[lb]).wait()
    pltpu.sync_copy(gbufs[lb],
                    uid_out.at[pl.ds(w * BW + (UID_SUB - 1) * SZ, SZ)])

    for b in range(NBUF):
        pltpu.async_copy(wi_hbm.at[hidx.at[b]], gbufs[b], gsems[b])

    def step(ti, _):
        for b in range(NBUF):
            t = ti * NBUF + b
            pltpu.make_async_copy(wi_hbm.at[pl.ds(0, SZ)], gbufs[b],
                                  gsems[b]).wait()
            pltpu.sync_copy(gbufs[b], acc.at[sidx.at[t]], add=True)
            nt = t + NBUF

            @pl.when(nt < NSUB)
            def _():
                pltpu.async_copy(wi_hbm.at[hidx.at[nt]], gbufs[b], gsems[b])
        return 0

    lax.fori_loop(0, NSUB // NBUF, step, 0)

    pltpu.sync_copy(acc.at[pl.ds(base_row, BW)],
                    sums_out.at[pl.ds(w * BW, BW)])


def _sc_lookup(uid2, hist2, sidx2, W_uid, W_item):
    mesh = plsc.VectorSubcoreMesh(core_axis_name="c", subcore_axis_name="s")
    return pl.kernel(
        _sc_body,
        out_type=(
            jax.ShapeDtypeStruct((B, D), jnp.float32),
            jax.ShapeDtypeStruct((B, D), jnp.float32),
        ),
        mesh=mesh,
        compiler_params=pltpu.CompilerParams(use_tc_tiling_on_sc=False),
        scratch_types=[
            pltpu.VMEM((UID_SUB, SZ), jnp.int32),
            pltpu.VMEM((NSUB, SZ), jnp.int32),
            pltpu.VMEM((NSUB, SZ), jnp.int32),
            [pltpu.VMEM((SZ, D), jnp.float32) for _ in range(NBUF)],
            pltpu.VMEM((SZ, D), jnp.float32),
            pltpu.VMEM_SHARED((NS * BW, D), jnp.float32),
            [pltpu.SemaphoreType.DMA for _ in range(NBUF)],
        ],
    )(uid2, hist2, sidx2, W_uid, W_item)


def _tc_body(hist_ref, sums_ref, w0_ref, out_ref):
    hist = hist_ref[...]
    cnt = jnp.sum((hist != 0).astype(jnp.float32), axis=1, keepdims=True)
    w0 = w0_ref[...]
    corrected = sums_ref[...] - (float(L) - cnt) * w0
    pooled = corrected / jnp.maximum(cnt, 1.0)
    out_ref[...] = jnp.where(cnt > 0.0, pooled, 0.0)


def _tc_combine(hist_item, sums, w0):
    blk = 2048
    return pl.pallas_call(
        _tc_body,
        grid=(B // blk,),
        in_specs=[
            pl.BlockSpec((blk, L), lambda i: (i, 0)),
            pl.BlockSpec((blk, D), lambda i: (i, 0)),
            pl.BlockSpec((1, D), lambda i: (0, 0)),
        ],
        out_specs=pl.BlockSpec((blk, D), lambda i: (i, 0)),
        out_shape=jax.ShapeDtypeStruct((B, D), jnp.float32),
    )(hist_item, sums, w0)


def kernel(uid, hist_item, W_uid, W_item):
    uid2 = uid.astype(jnp.int32).reshape(NW * UID_SUB, SZ)
    hist2 = hist_item.astype(jnp.int32).reshape(NW * NSUB, SZ)
    sidx2 = ((lax.iota(jnp.int32, B * L) // L) % (NS * BW)).reshape(
        NW * NSUB, SZ)
    uid_emb, sums = _sc_lookup(uid2, hist2, sidx2, W_uid, W_item)
    w0 = lax.slice(W_item, (0, 0), (1, D))
    pooled = _tc_combine(hist_item, sums, w0)
    return (uid_emb, pooled)


# speedup vs baseline: 1.6987x; 1.0001x over previous
import jax
import jax.numpy as jnp
from jax import lax
from jax.experimental import pallas as pl
from jax.experimental.pallas import tpu as pltpu
from jax.experimental.pallas import tpu_sc as plsc

B = 16384
L = 50
D = 32

NC = 2
NS = 16
NW = NC * NS
BW = B // NW
RW = BW * L
SZ = 128
NSUB = RW // SZ
NBUF = 8
UID_SUB = BW // SZ


def _sc_body(uid_hbm, hist_hbm, sidx_hbm, wu_hbm, wi_hbm, uid_out, sums_out,
             uidx, hidx, sidx, gbufs, zbuf, acc, gsems):
    c = lax.axis_index("c")
    s = lax.axis_index("s")
    w = c * NS + s
    base_row = s * BW

    pltpu.sync_copy(uid_hbm.at[pl.ds(w * UID_SUB, UID_SUB)], uidx)
    pltpu.sync_copy(hist_hbm.at[pl.ds(w * NSUB, NSUB)], hidx)
    pltpu.sync_copy(sidx_hbm.at[pl.ds(w * NSUB, NSUB)], sidx)

    zf = jnp.zeros((16,), jnp.float32)

    def zero(r, _):
        zbuf[r, pl.ds(0, 16)] = zf
        zbuf[r, pl.ds(16, 16)] = zf
        return 0

    lax.fori_loop(0, SZ, zero, 0)
    for k in range(BW // SZ):
        pltpu.sync_copy(zbuf, acc.at[pl.ds(base_row + k * SZ, SZ)])

    for k in range(UID_SUB):
        b = k % NBUF
        pltpu.async_copy(wu_hbm.at[uidx.at[k]], gbufs[b], gsems[b])
        if k >= 1:
            pb = (k - 1) % NBUF
            pltpu.make_async_copy(wu_hbm.at[pl.ds(0, SZ)], gbufs[pb],
                                  gsems[pb]).wait()
            pltpu.sync_copy(gbufs[pb],
                            uid_out.at[pl.ds(w * BW + (k - 1) * SZ, SZ)])
    lb = (UID_SUB - 1) % NBUF
    pltpu.make_async_copy(wu_hbm.at[pl.ds(0, SZ)], gbufs[lb], gsems[lb]).wait()
    pltpu.sync_copy(gbufs[lb],
                    uid_out.at[pl.ds(w * BW + (UID_SUB - 1) * SZ, SZ)])

    for b in range(NBUF):
        pltpu.async_copy(wi_hbm.at[hidx.at[b]], gbufs[b], gsems[b])

    def step(ti, _):
        for b in range(NBUF):
            t = ti * NBUF + b
            pltpu.make_async_copy(wi_hbm.at[pl.ds(0, SZ)], gbufs[b],
                                  gsems[b]).wait()
            pltpu.sync_copy(gbufs[b], acc.at[sidx.at[t]], add=True)
            nt = t + NBUF

            @pl.when(nt < NSUB)
            def _():
                pltpu.async_copy(wi_hbm.at[hidx.at[nt]], gbufs[b], gsems[b])
        return 0

    lax.fori_loop(0, NSUB // NBUF, step, 0)

    pltpu.sync_copy(zbuf, acc.at[sidx.at[0]], add=True)
    pltpu.sync_copy(zbuf, acc.at[sidx.at[0]], add=True)

    pltpu.sync_copy(acc.at[pl.ds(base_row, BW)],
                    sums_out.at[pl.ds(w * BW, BW)])


def _sc_lookup(uid2, hist2, sidx2, W_uid, W_item):
    mesh = plsc.VectorSubcoreMesh(core_axis_name="c", subcore_axis_name="s")
    return pl.kernel(
        _sc_body,
        out_type=(
            jax.ShapeDtypeStruct((B, D), jnp.float32),
            jax.ShapeDtypeStruct((B, D), jnp.float32),
        ),
        mesh=mesh,
        compiler_params=pltpu.CompilerParams(use_tc_tiling_on_sc=False),
        scratch_types=[
            pltpu.VMEM((UID_SUB, SZ), jnp.int32),
            pltpu.VMEM((NSUB, SZ), jnp.int32),
            pltpu.VMEM((NSUB, SZ), jnp.int32),
            [pltpu.VMEM((SZ, D), jnp.float32) for _ in range(NBUF)],
            pltpu.VMEM((SZ, D), jnp.float32),
            pltpu.VMEM_SHARED((NS * BW, D), jnp.float32),
            [pltpu.SemaphoreType.DMA for _ in range(NBUF)],
        ],
    )(uid2, hist2, sidx2, W_uid, W_item)


def _tc_body(hist_ref, sums_ref, w0_ref, out_ref):
    hist = hist_ref[...]
    cnt = jnp.sum((hist != 0).astype(jnp.float32), axis=1, keepdims=True)
    w0 = w0_ref[...]
    corrected = sums_ref[...] - (float(L) - cnt) * w0
    pooled = corrected / jnp.maximum(cnt, 1.0)
    out_ref[...] = jnp.where(cnt > 0.0, pooled, 0.0)


def _tc_combine(hist_item, sums, w0):
    blk = 2048
    return pl.pallas_call(
        _tc_body,
        grid=(B // blk,),
        in_specs=[
            pl.BlockSpec((blk, L), lambda i: (i, 0)),
            pl.BlockSpec((blk, D), lambda i: (i, 0)),
            pl.BlockSpec((1, D), lambda i: (0, 0)),
        ],
        out_specs=pl.BlockSpec((blk, D), lambda i: (i, 0)),
        out_shape=jax.ShapeDtypeStruct((B, D), jnp.float32),
    )(hist_item, sums, w0)


def kernel(uid, hist_item, W_uid, W_item):
    uid2 = uid.astype(jnp.int32).reshape(NW * UID_SUB, SZ)
    hist2 = hist_item.astype(jnp.int32).reshape(NW * NSUB, SZ)
    sidx2 = ((lax.iota(jnp.int32, B * L) // L) % (NS * BW)).reshape(
        NW * NSUB, SZ)
    uid_emb, sums = _sc_lookup(uid2, hist2, sidx2, W_uid, W_item)
    w0 = lax.slice(W_item, (0, 0), (1, D))
    pooled = _tc_combine(hist_item, sums, w0)
    return (uid_emb, pooled)
